# Initial kernel scaffold; baseline (speedup 1.0000x reference)
import jax
import jax.numpy as jnp
from jax.experimental import pallas as pl


def kernel(x, edge_index, edge_weight, W, b):
    raise NotImplementedError("write your pallas kernel here")


# speedup vs baseline: 12.0632x; 12.0632x over previous
import functools

import jax
import jax.numpy as jnp
from jax import lax
from jax.experimental import pallas as pl
from jax.experimental.pallas import tpu as pltpu
from jax.experimental.pallas import tpu_sc as plsc

N = 10000
E = 320000
C = 4
D = 128
NPAD = 10240
RPT = NPAD // 16
EPT_A = E // 32
EPT_C = E // 16
G = 16
DH = 64
NG = EPT_C // G
NBUF = 5
NH = NG // NBUF

_ZK = 25


@functools.cache
def _build_deg_kernel():
    mesh = plsc.VectorSubcoreMesh(
        core_axis_name="c", subcore_axis_name="s",
        num_cores=2, num_subcores=16)
    return functools.partial(
        pl.kernel,
        out_type=jax.ShapeDtypeStruct((2, NPAD), jnp.float32),
        mesh=mesh,
        scratch_types=[
            pltpu.VMEM((EPT_A,), jnp.int32),
            pltpu.VMEM((EPT_A,), jnp.float32),
            pltpu.VMEM((RPT,), jnp.float32),
            pltpu.VMEM_SHARED((NPAD,), jnp.float32),
            pltpu.SemaphoreType.DMA,
        ],
        compiler_params=pltpu.CompilerParams(needs_layout_passes=False, use_tc_tiling_on_sc=False),
    )(_deg_body)


def _deg_body(dst_hbm, w_hbm, degp_hbm, dst_v, w_v, zbuf, acc, sem):
    ci = lax.axis_index("c")
    si = lax.axis_index("s")
    base = (ci * 16 + si) * EPT_A
    pltpu.sync_copy(dst_hbm.at[pl.ds(base, EPT_A)], dst_v)
    pltpu.sync_copy(w_hbm.at[pl.ds(base, EPT_A)], w_v)
    zeros = jnp.zeros((16,), jnp.float32)

    def _zero(i, carry):
        zbuf[pl.ds(i * 16, 16)] = zeros
        return carry

    lax.fori_loop(0, RPT // 16, _zero, 0)
    pltpu.sync_copy(zbuf, acc.at[pl.ds(si * RPT, RPT)])
    plsc.subcore_barrier()

    def _super(h, carry):
        for b in range(_ZK):
            off = (h * _ZK + b) * G
            d16 = dst_v[pl.ds(off, G)]
            pltpu.async_copy(w_v.at[pl.ds(off, G)], acc.at[d16], sem, add=True)
        for b in range(_ZK):
            off = (h * _ZK + b) * G
            d16 = dst_v[pl.ds(off, G)]
            pltpu.make_async_copy(w_v.at[pl.ds(off, G)], acc.at[d16], sem).wait()
        return carry

    lax.fori_loop(0, (EPT_A // G) // _ZK, _super, 0)
    plsc.subcore_barrier()
    pltpu.sync_copy(acc.at[pl.ds(si * RPT, RPT)],
                    degp_hbm.at[ci, pl.ds(si * RPT, RPT)])


def _prep_body(degs_ref, x_ref, dis_ref, init_ref):
    deg = degs_ref[:, 0:1] + degs_ref[:, 1:2] + 1.0
    dis = lax.rsqrt(deg)
    dis_ref[...] = dis
    init_ref[...] = x_ref[...] * (dis * dis)


_BN_B = 1024
_prep_kernel = pl.pallas_call(
    _prep_body,
    grid=(NPAD // _BN_B,),
    in_specs=[pl.BlockSpec((_BN_B, 2), lambda i: (i, 0)),
              pl.BlockSpec((_BN_B, C * D), lambda i: (i, 0))],
    out_specs=[pl.BlockSpec((_BN_B, 1), lambda i: (i, 0)),
               pl.BlockSpec((_BN_B, C * D), lambda i: (i, 0))],
    out_shape=[jax.ShapeDtypeStruct((NPAD, 1), jnp.float32),
               jax.ShapeDtypeStruct((NPAD, C * D), jnp.float32)],
)


@functools.cache
def _build_msg_kernel():
    mesh = plsc.VectorSubcoreMesh(
        core_axis_name="c", subcore_axis_name="s",
        num_cores=2, num_subcores=16)
    return functools.partial(
        pl.kernel,
        out_type=jax.ShapeDtypeStruct((C, NPAD, D), jnp.float32),
        mesh=mesh,
        scratch_types=[
            pltpu.VMEM((EPT_C,), jnp.int32),
            pltpu.VMEM((EPT_C,), jnp.int32),
            pltpu.VMEM((EPT_C,), jnp.float32),
            pltpu.VMEM((NPAD,), jnp.float32),
            pltpu.VMEM((NBUF, G, DH), jnp.float32),
            pltpu.VMEM_SHARED((NPAD, DH), jnp.float32),
            [pltpu.SemaphoreType.DMA] * NBUF,
            [pltpu.SemaphoreType.DMA] * NBUF,
        ],
        compiler_params=pltpu.CompilerParams(needs_layout_passes=False, use_tc_tiling_on_sc=False),
    )(_msg_body)


def _msg_body(xr_hbm, init_hbm, dis_hbm, src_hbm, dst_hbm, w_hbm,
                agg_hbm, src_v, dst_v, w_v, dis_v, rowbuf, acc, gsems, ssems):
    ci = lax.axis_index("c")
    si = lax.axis_index("s")
    ebase = si * EPT_C
    nbase = si * RPT
    pltpu.sync_copy(src_hbm.at[pl.ds(ebase, EPT_C)], src_v)
    pltpu.sync_copy(dst_hbm.at[pl.ds(ebase, EPT_C)], dst_v)
    pltpu.sync_copy(w_hbm.at[pl.ds(ebase, EPT_C)], w_v)
    pltpu.sync_copy(dis_hbm, dis_v)

    def _pass(ph, carry):
        channel = ci * 2 + ph // 2
        half = ph % 2

        def _gidx(g):
            return src_v[pl.ds(g * G, G)] * (2 * C) + (channel * 2 + half)

        pltpu.sync_copy(
            init_hbm.at[pl.ds(nbase, RPT), pl.ds(channel * D + half * DH, DH)],
            acc.at[pl.ds(nbase, RPT)])
        plsc.subcore_barrier()
        for b in range(NBUF):
            pltpu.async_copy(xr_hbm.at[_gidx(b)], rowbuf.at[b], gsems[b])

        def _outer(h, carry2):
            for b in range(NBUF):
                g = h * NBUF + b
                off = g * G
                s16 = src_v[pl.ds(off, G)]
                d16 = dst_v[pl.ds(off, G)]
                w16 = w_v[pl.ds(off, G)]
                pltpu.make_async_copy(
                    xr_hbm.at[s16 * (2 * C) + (channel * 2 + half)],
                    rowbuf.at[b], gsems[b]).wait()
                n16 = (plsc.load_gather(dis_v, [s16]) * w16
                       * plsc.load_gather(dis_v, [d16]))
                for r in range(G):
                    sc = n16[r]
                    for j in range(DH // 16):
                        rowbuf[b, r, pl.ds(j * 16, 16)] = (
                            rowbuf[b, r, pl.ds(j * 16, 16)] * sc)
                pltpu.async_copy(rowbuf.at[b], acc.at[d16], ssems[b],
                                 add=True)
            for b in range(NBUF):
                @pl.when(h < NH - 1)
                def _refill():
                    off_p = (h * NBUF + b) * G
                    d16p = dst_v[pl.ds(off_p, G)]
                    pltpu.make_async_copy(rowbuf.at[b], acc.at[d16p],
                                          ssems[b]).wait()
                    g2 = (h + 1) * NBUF + b
                    pltpu.async_copy(xr_hbm.at[_gidx(g2)],
                                     rowbuf.at[b], gsems[b])
            return carry2

        lax.fori_loop(0, NH, _outer, 0)
        for b in range(NBUF):
            off = ((NH - 1) * NBUF + b) * G
            d16 = dst_v[pl.ds(off, G)]
            pltpu.make_async_copy(rowbuf.at[b], acc.at[d16], ssems[b]).wait()
        plsc.subcore_barrier()
        pltpu.sync_copy(
            acc.at[pl.ds(nbase, RPT)],
            agg_hbm.at[channel, pl.ds(nbase, RPT), pl.ds(half * DH, DH)])
        plsc.subcore_barrier()
        return carry

    lax.fori_loop(0, 2 * (C // 2), _pass, 0)


def _mm_body(agg_ref, w_ref, b_ref, out_ref):
    out_ref[0] = (jnp.dot(agg_ref[0], w_ref[0],
                          preferred_element_type=jnp.float32) + b_ref[0])


_BN_D = 512
_mm_kernel = pl.pallas_call(
    _mm_body,
    grid=(C, NPAD // _BN_D),
    in_specs=[pl.BlockSpec((1, _BN_D, D), lambda c, i: (c, i, 0)),
              pl.BlockSpec((1, D, D), lambda c, i: (c, 0, 0)),
              pl.BlockSpec((1, 1, D), lambda c, i: (c, 0, 0))],
    out_specs=pl.BlockSpec((1, _BN_D, D), lambda c, i: (c, i, 0)),
    out_shape=jax.ShapeDtypeStruct((C, NPAD, D), jnp.float32),
)


def kernel(x, edge_index, edge_weight, W, b):
    src = edge_index[0]
    dst = edge_index[1]
    x2 = jnp.pad(x.reshape(N, C * D), ((0, NPAD - N), (0, 0)))
    degp = _build_deg_kernel()(dst, edge_weight)
    dis2, init = _prep_kernel(degp.T, x2)
    agg = _build_msg_kernel()(x2.reshape(NPAD * C * 2, DH), init,
                              dis2.reshape(NPAD), src, dst, edge_weight)
    out4 = _mm_kernel(agg, W, b.reshape(C, 1, D))
    return out4.transpose(1, 0, 2)[:N]


# speedup vs baseline: 14.4998x; 1.2020x over previous
import functools

import jax
import jax.numpy as jnp
from jax import lax
from jax.experimental import pallas as pl
from jax.experimental.pallas import tpu as pltpu
from jax.experimental.pallas import tpu_sc as plsc

N = 10000
E = 320000
C = 4
D = 128
NPAD = 10240
RPT = NPAD // 16
EPT_A = E // 32
EPT_C = E // 16
G = 16
DH = 64
NG = EPT_C // G
GB = 80
NGB = EPT_C // GB
NBUF = 2
NHB = NGB // NBUF

_ZK = 25


@functools.cache
def _build_deg_kernel():
    mesh = plsc.VectorSubcoreMesh(
        core_axis_name="c", subcore_axis_name="s",
        num_cores=2, num_subcores=16)
    return functools.partial(
        pl.kernel,
        out_type=jax.ShapeDtypeStruct((2, NPAD), jnp.float32),
        mesh=mesh,
        scratch_types=[
            pltpu.VMEM((EPT_A,), jnp.int32),
            pltpu.VMEM((EPT_A,), jnp.float32),
            pltpu.VMEM((RPT,), jnp.float32),
            pltpu.VMEM_SHARED((NPAD,), jnp.float32),
            pltpu.SemaphoreType.DMA,
        ],
        compiler_params=pltpu.CompilerParams(needs_layout_passes=False, use_tc_tiling_on_sc=False),
    )(_deg_body)


def _deg_body(dst_hbm, w_hbm, degp_hbm, dst_v, w_v, zbuf, acc, sem):
    ci = lax.axis_index("c")
    si = lax.axis_index("s")
    base = (ci * 16 + si) * EPT_A
    pltpu.sync_copy(dst_hbm.at[pl.ds(base, EPT_A)], dst_v)
    pltpu.sync_copy(w_hbm.at[pl.ds(base, EPT_A)], w_v)
    zeros = jnp.zeros((16,), jnp.float32)

    def _zero(i, carry):
        zbuf[pl.ds(i * 16, 16)] = zeros
        return carry

    lax.fori_loop(0, RPT // 16, _zero, 0)
    pltpu.sync_copy(zbuf, acc.at[pl.ds(si * RPT, RPT)])
    plsc.subcore_barrier()

    def _super(h, carry):
        for b in range(_ZK):
            off = (h * _ZK + b) * G
            d16 = dst_v[pl.ds(off, G)]
            pltpu.async_copy(w_v.at[pl.ds(off, G)], acc.at[d16], sem, add=True)
        for b in range(_ZK):
            off = (h * _ZK + b) * G
            d16 = dst_v[pl.ds(off, G)]
            pltpu.make_async_copy(w_v.at[pl.ds(off, G)], acc.at[d16], sem).wait()
        return carry

    lax.fori_loop(0, (EPT_A // G) // _ZK, _super, 0)
    plsc.subcore_barrier()
    pltpu.sync_copy(acc.at[pl.ds(si * RPT, RPT)],
                    degp_hbm.at[ci, pl.ds(si * RPT, RPT)])


def _prep_body(degs_ref, x_ref, dis_ref, init_ref):
    deg = degs_ref[:, 0:1] + degs_ref[:, 1:2] + 1.0
    dis = lax.rsqrt(deg)
    dis_ref[...] = dis
    init_ref[...] = x_ref[...] * (dis * dis)


_BN_B = 1024
_prep_kernel = pl.pallas_call(
    _prep_body,
    grid=(NPAD // _BN_B,),
    in_specs=[pl.BlockSpec((_BN_B, 2), lambda i: (i, 0)),
              pl.BlockSpec((_BN_B, C * D), lambda i: (i, 0))],
    out_specs=[pl.BlockSpec((_BN_B, 1), lambda i: (i, 0)),
               pl.BlockSpec((_BN_B, C * D), lambda i: (i, 0))],
    out_shape=[jax.ShapeDtypeStruct((NPAD, 1), jnp.float32),
               jax.ShapeDtypeStruct((NPAD, C * D), jnp.float32)],
)


@functools.cache
def _build_msg_kernel():
    mesh = plsc.VectorSubcoreMesh(
        core_axis_name="c", subcore_axis_name="s",
        num_cores=2, num_subcores=16)
    return functools.partial(
        pl.kernel,
        out_type=jax.ShapeDtypeStruct((C, NPAD, D), jnp.float32),
        mesh=mesh,
        scratch_types=[
            pltpu.VMEM((EPT_C,), jnp.int32),
            pltpu.VMEM((EPT_C,), jnp.int32),
            pltpu.VMEM((EPT_C,), jnp.float32),
            pltpu.VMEM((NPAD,), jnp.float32),
            pltpu.VMEM((NBUF, GB, DH), jnp.float32),
            pltpu.VMEM_SHARED((NPAD, DH), jnp.float32),
            [pltpu.SemaphoreType.DMA] * NBUF,
            [pltpu.SemaphoreType.DMA] * NBUF,
        ],
        compiler_params=pltpu.CompilerParams(needs_layout_passes=False, use_tc_tiling_on_sc=False),
    )(_msg_body)


def _msg_body(xr_hbm, init_hbm, dis_hbm, src_hbm, dst_hbm, w_hbm,
              agg_hbm, src_v, dst_v, w_v, dis_v, rowbuf, acc,
              gsems, ssems):
    ci = lax.axis_index("c")
    si = lax.axis_index("s")
    ebase = si * EPT_C
    nbase = si * RPT
    pltpu.sync_copy(src_hbm.at[pl.ds(ebase, EPT_C)], src_v)
    pltpu.sync_copy(dst_hbm.at[pl.ds(ebase, EPT_C)], dst_v)
    pltpu.sync_copy(w_hbm.at[pl.ds(ebase, EPT_C)], w_v)
    pltpu.sync_copy(dis_hbm, dis_v)

    def _pass(ph, carry):
        channel = ci * 2 + ph // 2
        half = ph % 2
        q = channel * 2 + half

        pltpu.sync_copy(
            init_hbm.at[pl.ds(nbase, RPT), pl.ds(channel * D + half * DH, DH)],
            acc.at[pl.ds(nbase, RPT)])
        plsc.subcore_barrier()

        def _gather(g, b):
            for sub in range(GB // G):
                off = g * GB + sub * G
                i16 = src_v[pl.ds(off, G)] * (2 * C) + q
                pltpu.async_copy(xr_hbm.at[i16],
                                 rowbuf.at[b].at[pl.ds(sub * G, G)],
                                 gsems[b])

        def _scat_desc(g, b, sub):
            d16 = dst_v[pl.ds(g * GB + sub * G, G)]
            return pltpu.make_async_copy(
                rowbuf.at[b].at[pl.ds(sub * G, G)], acc.at[d16], ssems[b])

        for b in range(NBUF):
            _gather(b, b)

        def _outer(h, carry2):
            for b in range(NBUF):
                g = h * NBUF + b
                for sub in range(GB // G):
                    off0 = g * GB + sub * G
                    i16 = src_v[pl.ds(off0, G)] * (2 * C) + q
                    pltpu.make_async_copy(xr_hbm.at[i16],
                                          rowbuf.at[b].at[pl.ds(sub * G, G)],
                                          gsems[b]).wait()
                for sub in range(GB // G):
                    off = g * GB + sub * G
                    s16 = src_v[pl.ds(off, G)]
                    d16 = dst_v[pl.ds(off, G)]
                    w16 = w_v[pl.ds(off, G)]
                    n16 = (plsc.load_gather(dis_v, [s16]) * w16
                           * plsc.load_gather(dis_v, [d16]))
                    for r in range(G):
                        sc = n16[r]
                        rr = sub * G + r
                        for j in range(DH // 16):
                            rowbuf[b, rr, pl.ds(j * 16, 16)] = (
                                rowbuf[b, rr, pl.ds(j * 16, 16)] * sc)
                for sub in range(GB // G):
                    d16 = dst_v[pl.ds(g * GB + sub * G, G)]
                    pltpu.async_copy(rowbuf.at[b].at[pl.ds(sub * G, G)],
                                     acc.at[d16], ssems[b], add=True)
            for b in range(NBUF):
                @pl.when(h < NHB - 1)
                def _refill():
                    for sub in range(GB // G):
                        _scat_desc(h * NBUF + b, b, sub).wait()
                    _gather((h + 1) * NBUF + b, b)
            return carry2

        lax.fori_loop(0, NHB, _outer, 0)
        for b in range(NBUF):
            for sub in range(GB // G):
                _scat_desc((NHB - 1) * NBUF + b, b, sub).wait()
        plsc.subcore_barrier()
        pltpu.sync_copy(
            acc.at[pl.ds(nbase, RPT)],
            agg_hbm.at[channel, pl.ds(nbase, RPT), pl.ds(half * DH, DH)])
        plsc.subcore_barrier()
        return carry

    lax.fori_loop(0, 2 * (C // 2), _pass, 0)


def _mm_body(agg_ref, w_ref, b_ref, out_ref):
    out_ref[0] = (jnp.dot(agg_ref[0], w_ref[0],
                          preferred_element_type=jnp.float32) + b_ref[0])


_BN_D = 512
_mm_kernel = pl.pallas_call(
    _mm_body,
    grid=(C, NPAD // _BN_D),
    in_specs=[pl.BlockSpec((1, _BN_D, D), lambda c, i: (c, i, 0)),
              pl.BlockSpec((1, D, D), lambda c, i: (c, 0, 0)),
              pl.BlockSpec((1, 1, D), lambda c, i: (c, 0, 0))],
    out_specs=pl.BlockSpec((1, _BN_D, D), lambda c, i: (c, i, 0)),
    out_shape=jax.ShapeDtypeStruct((C, NPAD, D), jnp.float32),
)


def kernel(x, edge_index, edge_weight, W, b):
    src = edge_index[0]
    dst = edge_index[1]
    x2 = jnp.pad(x.reshape(N, C * D), ((0, NPAD - N), (0, 0)))
    degp = _build_deg_kernel()(dst, edge_weight)
    dis2, init = _prep_kernel(degp.T, x2)
    agg = _build_msg_kernel()(x2.reshape(NPAD * C * 2, DH), init,
                              dis2.reshape(NPAD), src, dst, edge_weight)
    out4 = _mm_kernel(agg, W, b.reshape(C, 1, D))
    return out4.transpose(1, 0, 2)[:N]


# speedup vs baseline: 15.1405x; 1.0442x over previous
import functools

import jax
import jax.numpy as jnp
from jax import lax
from jax.experimental import pallas as pl
from jax.experimental.pallas import tpu as pltpu
from jax.experimental.pallas import tpu_sc as plsc

N = 10000
E = 320000
C = 4
D = 128
NPAD = 10240
RPT = NPAD // 16
EPT_A = E // 32
EPT_C = E // 16
G = 16
DH = 64
NG = EPT_C // G
GB = 80
NGB = EPT_C // GB
NBUF = 2
NHB = NGB // NBUF

_ZK = 25


@functools.cache
def _build_deg_kernel():
    mesh = plsc.VectorSubcoreMesh(
        core_axis_name="c", subcore_axis_name="s",
        num_cores=2, num_subcores=16)
    return functools.partial(
        pl.kernel,
        out_type=jax.ShapeDtypeStruct((2, NPAD), jnp.float32),
        mesh=mesh,
        scratch_types=[
            pltpu.VMEM((EPT_A,), jnp.int32),
            pltpu.VMEM((EPT_A,), jnp.float32),
            pltpu.VMEM((RPT,), jnp.float32),
            pltpu.VMEM_SHARED((NPAD,), jnp.float32),
            pltpu.SemaphoreType.DMA,
        ],
        compiler_params=pltpu.CompilerParams(needs_layout_passes=False, use_tc_tiling_on_sc=False),
    )(_deg_body)


def _deg_body(dst_hbm, w_hbm, degp_hbm, dst_v, w_v, zbuf, acc, sem):
    ci = lax.axis_index("c")
    si = lax.axis_index("s")
    base = (ci * 16 + si) * EPT_A
    pltpu.sync_copy(dst_hbm.at[pl.ds(base, EPT_A)], dst_v)
    pltpu.sync_copy(w_hbm.at[pl.ds(base, EPT_A)], w_v)
    zeros = jnp.zeros((16,), jnp.float32)

    def _zero(i, carry):
        zbuf[pl.ds(i * 16, 16)] = zeros
        return carry

    lax.fori_loop(0, RPT // 16, _zero, 0)
    pltpu.sync_copy(zbuf, acc.at[pl.ds(si * RPT, RPT)])
    plsc.subcore_barrier()

    def _super(h, carry):
        for b in range(_ZK):
            off = (h * _ZK + b) * G
            d16 = dst_v[pl.ds(off, G)]
            pltpu.async_copy(w_v.at[pl.ds(off, G)], acc.at[d16], sem, add=True)
        for b in range(_ZK):
            off = (h * _ZK + b) * G
            d16 = dst_v[pl.ds(off, G)]
            pltpu.make_async_copy(w_v.at[pl.ds(off, G)], acc.at[d16], sem).wait()
        return carry

    lax.fori_loop(0, (EPT_A // G) // _ZK, _super, 0)
    plsc.subcore_barrier()
    pltpu.sync_copy(acc.at[pl.ds(si * RPT, RPT)],
                    degp_hbm.at[ci, pl.ds(si * RPT, RPT)])


def _prep_body(degs_ref, x_ref, dis_ref, init_ref):
    deg = degs_ref[:, 0:1] + degs_ref[:, 1:2] + 1.0
    dis = lax.rsqrt(deg)
    dis_ref[...] = dis
    init_ref[...] = x_ref[...] * (dis * dis)


_BN_B = 1024
_prep_kernel = pl.pallas_call(
    _prep_body,
    grid=(NPAD // _BN_B,),
    in_specs=[pl.BlockSpec((_BN_B, 2), lambda i: (i, 0)),
              pl.BlockSpec((_BN_B, C * D), lambda i: (i, 0))],
    out_specs=[pl.BlockSpec((_BN_B, 1), lambda i: (i, 0)),
               pl.BlockSpec((_BN_B, C * D), lambda i: (i, 0))],
    out_shape=[jax.ShapeDtypeStruct((NPAD, 1), jnp.float32),
               jax.ShapeDtypeStruct((NPAD, C * D), jnp.float32)],
)


@functools.cache
def _build_msg_kernel():
    mesh = plsc.VectorSubcoreMesh(
        core_axis_name="c", subcore_axis_name="s",
        num_cores=2, num_subcores=16)
    return functools.partial(
        pl.kernel,
        out_type=jax.ShapeDtypeStruct((C, NPAD, D), jnp.float32),
        mesh=mesh,
        scratch_types=[
            pltpu.VMEM((EPT_C,), jnp.int32),
            pltpu.VMEM((EPT_C,), jnp.int32),
            pltpu.VMEM((EPT_C,), jnp.float32),
            pltpu.VMEM((NPAD,), jnp.float32),
            pltpu.VMEM((NBUF, GB, DH), jnp.float32),
            pltpu.VMEM_SHARED((NPAD, DH), jnp.float32),
            [pltpu.SemaphoreType.DMA] * NBUF,
            [pltpu.SemaphoreType.DMA] * NBUF,
        ],
        compiler_params=pltpu.CompilerParams(needs_layout_passes=False, use_tc_tiling_on_sc=False),
    )(_msg_body)


def _msg_body(xr_hbm, init_hbm, dis_hbm, src_hbm, dst_hbm, w_hbm,
              agg_hbm, src_v, dst_v, w_v, dis_v, rowbuf, acc,
              gsems, ssems):
    ci = lax.axis_index("c")
    si = lax.axis_index("s")
    ebase = si * EPT_C
    nbase = si * RPT
    pltpu.sync_copy(src_hbm.at[pl.ds(ebase, EPT_C)], src_v)
    pltpu.sync_copy(dst_hbm.at[pl.ds(ebase, EPT_C)], dst_v)
    pltpu.sync_copy(w_hbm.at[pl.ds(ebase, EPT_C)], w_v)
    pltpu.sync_copy(dis_hbm, dis_v)

    def _pass(ph, carry):
        channel = ci * 2 + ph // 2
        half = ph % 2
        q = channel * 2 + half

        pltpu.sync_copy(
            init_hbm.at[pl.ds(nbase, RPT), pl.ds(channel * D + half * DH, DH)],
            acc.at[pl.ds(nbase, RPT)])
        plsc.subcore_barrier()

        def _gather(g, b):
            for sub in range(GB // G):
                off = g * GB + sub * G
                i16 = src_v[pl.ds(off, G)] * (2 * C) + q
                pltpu.async_copy(xr_hbm.at[i16],
                                 rowbuf.at[b].at[pl.ds(sub * G, G)],
                                 gsems[b])

        def _scat_desc(g, b, sub):
            d16 = dst_v[pl.ds(g * GB + sub * G, G)]
            return pltpu.make_async_copy(
                rowbuf.at[b].at[pl.ds(sub * G, G)], acc.at[d16], ssems[b])

        for b in range(NBUF):
            _gather(b, b)

        def _outer(h, carry2):
            for b in range(NBUF):
                g = h * NBUF + b
                norms = []
                for sub in range(GB // G):
                    off = g * GB + sub * G
                    s16 = src_v[pl.ds(off, G)]
                    w16 = w_v[pl.ds(off, G)]
                    d16 = dst_v[pl.ds(off, G)]
                    norms.append(plsc.load_gather(dis_v, [s16]) * w16
                                 * plsc.load_gather(dis_v, [d16]))
                for sub in range(GB // G):
                    off0 = g * GB + sub * G
                    i16 = src_v[pl.ds(off0, G)] * (2 * C) + q
                    pltpu.make_async_copy(xr_hbm.at[i16],
                                          rowbuf.at[b].at[pl.ds(sub * G, G)],
                                          gsems[b]).wait()
                for sub in range(GB // G):
                    off = g * GB + sub * G
                    d16 = dst_v[pl.ds(off, G)]
                    n16 = norms[sub]
                    for r in range(G):
                        sc = n16[r]
                        rr = sub * G + r
                        for j in range(DH // 16):
                            rowbuf[b, rr, pl.ds(j * 16, 16)] = (
                                rowbuf[b, rr, pl.ds(j * 16, 16)] * sc)
                for sub in range(GB // G):
                    d16 = dst_v[pl.ds(g * GB + sub * G, G)]
                    pltpu.async_copy(rowbuf.at[b].at[pl.ds(sub * G, G)],
                                     acc.at[d16], ssems[b], add=True)
            for b in range(NBUF):
                @pl.when(h < NHB - 1)
                def _refill():
                    for sub in range(GB // G):
                        _scat_desc(h * NBUF + b, b, sub).wait()
                    _gather((h + 1) * NBUF + b, b)
            return carry2

        lax.fori_loop(0, NHB, _outer, 0)
        for b in range(NBUF):
            for sub in range(GB // G):
                _scat_desc((NHB - 1) * NBUF + b, b, sub).wait()
        plsc.subcore_barrier()
        pltpu.sync_copy(
            acc.at[pl.ds(nbase, RPT)],
            agg_hbm.at[channel, pl.ds(nbase, RPT), pl.ds(half * DH, DH)])
        plsc.subcore_barrier()
        return carry

    lax.fori_loop(0, 2 * (C // 2), _pass, 0)


def _mm_body(agg_ref, w_ref, b_ref, out_ref):
    out_ref[0] = (jnp.dot(agg_ref[0], w_ref[0],
                          preferred_element_type=jnp.float32) + b_ref[0])


_BN_D = 512
_mm_kernel = pl.pallas_call(
    _mm_body,
    grid=(C, NPAD // _BN_D),
    in_specs=[pl.BlockSpec((1, _BN_D, D), lambda c, i: (c, i, 0)),
              pl.BlockSpec((1, D, D), lambda c, i: (c, 0, 0)),
              pl.BlockSpec((1, 1, D), lambda c, i: (c, 0, 0))],
    out_specs=pl.BlockSpec((1, _BN_D, D), lambda c, i: (c, i, 0)),
    out_shape=jax.ShapeDtypeStruct((C, NPAD, D), jnp.float32),
)


def kernel(x, edge_index, edge_weight, W, b):
    src = edge_index[0]
    dst = edge_index[1]
    x2 = jnp.pad(x.reshape(N, C * D), ((0, NPAD - N), (0, 0)))
    degp = _build_deg_kernel()(dst, edge_weight)
    dis2, init = _prep_kernel(degp.T, x2)
    agg = _build_msg_kernel()(x2.reshape(NPAD * C * 2, DH), init,
                              dis2.reshape(NPAD), src, dst, edge_weight)
    out4 = _mm_kernel(agg, W, b.reshape(C, 1, D))
    return out4.transpose(1, 0, 2)[:N]
